# baseline (speedup 1.0000x reference)
import functools

import jax
import jax.numpy as jnp
from jax import lax
from jax.experimental import pallas as pl
from jax.experimental.pallas import tpu as pltpu
from jax.experimental.pallas import tpu_sc as plsc

NC = 2
NS = 16
NW = NC * NS
CHUNK = 128
NB = 4


def _mesh():
    return plsc.VectorSubcoreMesh(core_axis_name="c", subcore_axis_name="s")


def _pad_rows(N):
    r = -(-N // NS)
    return -(-r // 8) * 8


def _deg_call(n_chunks, N):
    n_outer = n_chunks // NB
    assert n_outer % 2 == 0 and n_outer >= 4
    rows_per_tile = _pad_rows(N)
    NP = rows_per_tile * NS

    idx_bufs = [pltpu.VMEM((CHUNK,), jnp.int32) for _ in range(2 * NB)]

    @functools.partial(
        pl.kernel,
        mesh=_mesh(),
        out_type=jax.ShapeDtypeStruct((NC, NP, 16), jnp.float32),
        compiler_params=pltpu.CompilerParams(use_tc_tiling_on_sc=False),
        scratch_types=idx_bufs + [
            pltpu.VMEM((CHUNK, 16), jnp.float32),
            pltpu.VMEM_SHARED((NP, 16), jnp.float32),
            pltpu.SemaphoreType.DMA((2 * NB,)),
            pltpu.SemaphoreType.DMA((NB,)),
        ],
    )
    def k(dsts_hbm, ones_hbm, zeros_hbm, out_hbm, *refs):
        db = [refs[0:NB], refs[NB:2 * NB]]
        ones_v, acc, sem_i, sem_s = refs[2 * NB:]
        c = lax.axis_index("c")
        s = lax.axis_index("s")
        wid = c * NS + s
        rbase = s * rows_per_tile
        pltpu.sync_copy(zeros_hbm, acc.at[pl.ds(rbase, rows_per_tile)])
        pltpu.sync_copy(ones_hbm, ones_v)
        plsc.subcore_barrier()

        def start_idx(i, p, b):
            pltpu.async_copy(dsts_hbm.at[wid, i], db[p][b],
                             sem_i.at[p * NB + b])

        def wait_idx(p, b):
            pltpu.make_async_copy(dsts_hbm.at[wid, 0], db[p][b],
                                  sem_i.at[p * NB + b]).wait()

        def start_scatter(p, b):
            pltpu.async_copy(ones_v, acc.at[db[p][b]], sem_s.at[b], add=True)

        def wait_scatter(b):
            pltpu.make_async_copy(ones_v, acc.at[db[0][b]],
                                  sem_s.at[b]).wait()

        for b in range(NB):
            start_idx(b, 0, b)
        for b in range(NB):
            wait_idx(0, b)
            start_scatter(0, b)
        for b in range(NB):
            start_idx(NB + b, 1, b)
        for b in range(NB):
            wait_idx(1, b)
            wait_scatter(b)
            start_scatter(1, b)
        for b in range(NB):
            start_idx(2 * NB + b, 0, b)

        def body(o2, carry):
            ga = 2 * o2
            for b in range(NB):
                wait_idx(0, b)
                wait_scatter(b)
                start_scatter(0, b)
            for b in range(NB):
                start_idx((ga + 1) * NB + b, 1, b)
            for b in range(NB):
                wait_idx(1, b)
                wait_scatter(b)
                start_scatter(1, b)
            for b in range(NB):
                start_idx((ga + 2) * NB + b, 0, b)
            return carry

        lax.fori_loop(1, n_outer // 2 - 1, body, 0)
        for b in range(NB):
            wait_idx(0, b)
            wait_scatter(b)
            start_scatter(0, b)
        for b in range(NB):
            start_idx((n_outer - 1) * NB + b, 1, b)
        for b in range(NB):
            wait_idx(1, b)
            wait_scatter(b)
            start_scatter(1, b)
        for b in range(NB):
            wait_scatter(b)

        plsc.subcore_barrier()
        pltpu.sync_copy(acc.at[pl.ds(rbase, rows_per_tile)],
                        out_hbm.at[c, pl.ds(rbase, rows_per_tile)])

    return k


def _agg_call(n_chunks, N, NGV, D):
    n_outer = n_chunks // NB
    assert n_outer % 2 == 0 and n_outer >= 4
    rows_per_tile = _pad_rows(N)
    NP = rows_per_tile * NS

    idx_bufs = [pltpu.VMEM((CHUNK,), jnp.int32) for _ in range(4 * NB)]
    row_bufs = [pltpu.VMEM((CHUNK, D), jnp.float32) for _ in range(NB)]

    @functools.partial(
        pl.kernel,
        mesh=_mesh(),
        out_type=jax.ShapeDtypeStruct((NC, NP, 128), jnp.float32),
        compiler_params=pltpu.CompilerParams(use_tc_tiling_on_sc=False),
        scratch_types=idx_bufs + row_bufs + [
            pltpu.VMEM_SHARED((NP, D), jnp.float32),
            pltpu.SemaphoreType.DMA((2 * NB,)),
            pltpu.SemaphoreType.DMA((NB,)),
            pltpu.SemaphoreType.DMA((NB,)),
        ],
    )
    def k(srcs_hbm, dsts_hbm, g_hbm, zeros_hbm, out_hbm, *refs):
        sb = [refs[0:NB], refs[NB:2 * NB]]
        db = [refs[2 * NB:3 * NB], refs[3 * NB:4 * NB]]
        rb = refs[4 * NB:5 * NB]
        acc, sem_i, sem_g, sem_s = refs[5 * NB:]
        c = lax.axis_index("c")
        s = lax.axis_index("s")
        wid = c * NS + s
        rbase = s * rows_per_tile
        pltpu.sync_copy(zeros_hbm, acc.at[pl.ds(rbase, rows_per_tile)])
        plsc.subcore_barrier()

        def start_idx(i, p, b):
            pltpu.async_copy(srcs_hbm.at[wid, i], sb[p][b],
                             sem_i.at[p * NB + b])
            pltpu.async_copy(dsts_hbm.at[wid, i], db[p][b],
                             sem_i.at[p * NB + b])

        def wait_idx(p, b):
            pltpu.make_async_copy(srcs_hbm.at[wid, 0], sb[p][b],
                                  sem_i.at[p * NB + b]).wait()
            pltpu.make_async_copy(dsts_hbm.at[wid, 0], db[p][b],
                                  sem_i.at[p * NB + b]).wait()

        def start_gather(p, b):
            pltpu.async_copy(g_hbm.at[sb[p][b]], rb[b], sem_g.at[b])

        def wait_gather(p, b):
            pltpu.make_async_copy(g_hbm.at[sb[p][b]], rb[b],
                                  sem_g.at[b]).wait()

        def start_scatter(p, b):
            pltpu.async_copy(rb[b], acc.at[db[p][b]], sem_s.at[b], add=True)

        def wait_scatter(b):
            pltpu.make_async_copy(rb[b], acc.at[db[0][b]],
                                  sem_s.at[b]).wait()

        for b in range(NB):
            start_idx(b, 0, b)
        for b in range(NB):
            wait_idx(0, b)
            start_gather(0, b)
        for b in range(NB):
            wait_gather(0, b)
            start_idx(NB + b, 1, b)
            start_scatter(0, b)
        for b in range(NB):
            wait_idx(1, b)
            wait_scatter(b)
            start_gather(1, b)
        for b in range(NB):
            wait_gather(1, b)
            start_idx(2 * NB + b, 0, b)
            start_scatter(1, b)

        def body(o2, carry):
            ga = 2 * o2
            for b in range(NB):
                wait_idx(0, b)
                wait_scatter(b)
                start_gather(0, b)
            for b in range(NB):
                wait_gather(0, b)
                start_idx((ga + 1) * NB + b, 1, b)
                start_scatter(0, b)
            for b in range(NB):
                wait_idx(1, b)
                wait_scatter(b)
                start_gather(1, b)
            for b in range(NB):
                wait_gather(1, b)
                start_idx((ga + 2) * NB + b, 0, b)
                start_scatter(1, b)
            return carry

        lax.fori_loop(1, n_outer // 2 - 1, body, 0)
        for b in range(NB):
            wait_idx(0, b)
            wait_scatter(b)
            start_gather(0, b)
        for b in range(NB):
            wait_gather(0, b)
            start_idx((n_outer - 1) * NB + b, 1, b)
            start_scatter(0, b)
        for b in range(NB):
            wait_idx(1, b)
            wait_scatter(b)
            start_gather(1, b)
        for b in range(NB):
            wait_gather(1, b)
            start_scatter(1, b)
        for b in range(NB):
            wait_scatter(b)

        plsc.subcore_barrier()
        pltpu.sync_copy(acc.at[pl.ds(rbase, rows_per_tile)],
                        out_hbm.at[c, pl.ds(rbase, rows_per_tile),
                                   pl.ds(0, D)])

    return k


def _matmul(x, W):
    N, K = x.shape
    H = W.shape[1]
    R = 2000

    def body(x_ref, w_ref, h_ref):
        h_ref[...] = jnp.dot(x_ref[...], w_ref[...],
                             preferred_element_type=jnp.float32)

    return pl.pallas_call(
        body,
        grid=(N // R,),
        in_specs=[
            pl.BlockSpec((R, K), lambda i: (i, 0)),
            pl.BlockSpec((K, H), lambda i: (0, 0)),
        ],
        out_specs=pl.BlockSpec((R, H), lambda i: (i, 0)),
        out_shape=jax.ShapeDtypeStruct((N, H), jnp.float32),
    )(x, W)


def _scale_g(h, deg2d):
    N, H = h.shape
    R = 2560
    grid = -(-N // R)

    def body(d_ref, h_ref, g_ref, dinv_ref):
        dsum = d_ref[0:1, :] + d_ref[1:2, :]
        dinv_col = lax.rsqrt(dsum + 1.0).T
        g_ref[:, 0:H] = h_ref[...] * dinv_col
        g_ref[:, H:128] = jnp.zeros((R, 128 - H), jnp.float32)
        dinv_ref[...] = dinv_col

    return pl.pallas_call(
        body,
        grid=(grid,),
        in_specs=[
            pl.BlockSpec((NC, R), lambda i: (0, i)),
            pl.BlockSpec((R, H), lambda i: (i, 0)),
        ],
        out_specs=[
            pl.BlockSpec((R, 128), lambda i: (i, 0)),
            pl.BlockSpec((R, 1), lambda i: (i, 0)),
        ],
        out_shape=[
            jax.ShapeDtypeStruct((N, 128), jnp.float32),
            jax.ShapeDtypeStruct((N, 1), jnp.float32),
        ],
    )(deg2d, h)


def _mid_layer(accp, g1p, dinv, b1, W2):
    N = g1p.shape[0]
    H = b1.shape[1]
    O = W2.shape[1]
    R = 2000

    def body(a0_ref, a1_ref, g1_ref, dinv_ref, b1_ref, w2_ref, g2_ref):
        dinv = dinv_ref[...]
        a0 = a0_ref[0, :, 0:H]
        a1 = a1_ref[0, :, 0:H]
        g1 = g1_ref[:, 0:H]
        out1 = dinv * (a0 + a1 + g1) + b1_ref[...]
        out1 = jnp.maximum(out1, 0.0)
        g2_ref[:, 0:O] = jnp.dot(out1, w2_ref[...],
                                 preferred_element_type=jnp.float32) * dinv
        g2_ref[:, O:128] = jnp.zeros((R, 128 - O), jnp.float32)

    return pl.pallas_call(
        body,
        grid=(N // R,),
        in_specs=[
            pl.BlockSpec((1, R, 128), lambda i: (0, i, 0)),
            pl.BlockSpec((1, R, 128), lambda i: (1, i, 0)),
            pl.BlockSpec((R, 128), lambda i: (i, 0)),
            pl.BlockSpec((R, 1), lambda i: (i, 0)),
            pl.BlockSpec((1, H), lambda i: (0, 0)),
            pl.BlockSpec((H, O), lambda i: (0, 0)),
        ],
        out_specs=pl.BlockSpec((R, 128), lambda i: (i, 0)),
        out_shape=jax.ShapeDtypeStruct((N, 128), jnp.float32),
    )(accp, accp, g1p, dinv, b1, W2)


def _final_layer(accp, g2p, dinv, b2):
    N = g2p.shape[0]
    O = b2.shape[1]
    R = 2000

    def body(c0_ref, c1_ref, g2_ref, dinv_ref, b2_ref, o_ref):
        o_ref[...] = dinv_ref[...] * (c0_ref[0, :, 0:O] + c1_ref[0, :, 0:O]
                                      + g2_ref[:, 0:O]) + b2_ref[...]

    return pl.pallas_call(
        body,
        grid=(N // R,),
        in_specs=[
            pl.BlockSpec((1, R, 128), lambda i: (0, i, 0)),
            pl.BlockSpec((1, R, 128), lambda i: (1, i, 0)),
            pl.BlockSpec((R, 128), lambda i: (i, 0)),
            pl.BlockSpec((R, 1), lambda i: (i, 0)),
            pl.BlockSpec((1, O), lambda i: (0, 0)),
        ],
        out_specs=pl.BlockSpec((R, O), lambda i: (i, 0)),
        out_shape=jax.ShapeDtypeStruct((N, O), jnp.float32),
    )(accp, accp, g2p, dinv, b2)


def kernel(x, edge_index, W1, b1, W2, b2):
    N, _ = x.shape
    E = edge_index.shape[1]
    H = W1.shape[1]
    O = W2.shape[1]
    rows_per_tile = _pad_rows(N)
    NP = rows_per_tile * NS

    quant = NW * CHUNK * 2 * NB
    epw = -(-E // quant) * (CHUNK * 2 * NB)
    n_chunks = epw // CHUNK
    pad_e = NW * epw - E
    if pad_e:
        pad_i = jnp.arange(pad_e, dtype=jnp.int32)
        src = jnp.concatenate([edge_index[0], pad_i % N])
        dst = jnp.concatenate([edge_index[1], N + pad_i % (NP - N)])
    else:
        src = edge_index[0]
        dst = edge_index[1]
    dsts = dst.reshape(NW, n_chunks, CHUNK)
    srcs_h = (src * (128 // H)).reshape(NW, n_chunks, CHUNK)
    srcs_o = (src * (128 // O)).reshape(NW, n_chunks, CHUNK)

    ones_c = jnp.ones((CHUNK, 16), jnp.float32)
    zeros_deg = jnp.zeros((rows_per_tile, 16), jnp.float32)
    deg_parts = _deg_call(n_chunks, N)(dsts, ones_c, zeros_deg)
    deg2d = deg_parts[:, :, 0]

    h1 = _matmul(x, W1)
    g1p, dinv = _scale_g(h1, deg2d)

    zeros_h = jnp.zeros((rows_per_tile, H), jnp.float32)
    g1v = g1p.reshape(N * (128 // H), H)
    acc1 = _agg_call(n_chunks, N, N * (128 // H), H)(srcs_h, dsts, g1v,
                                                     zeros_h)

    g2p = _mid_layer(acc1, g1p, dinv, b1.reshape(1, H), W2)

    zeros_o = jnp.zeros((rows_per_tile, O), jnp.float32)
    g2v = g2p.reshape(N * (128 // O), O)
    acc2 = _agg_call(n_chunks, N, N * (128 // O), O)(srcs_o, dsts, g2v,
                                                     zeros_o)

    return _final_layer(acc2, g2p, dinv, b2.reshape(1, O))
